# baseline (speedup 1.0000x reference)
import functools

import jax
import jax.numpy as jnp
from jax import lax
from jax.experimental import pallas as pl
from jax.experimental.pallas import tpu as pltpu
from jax.experimental.pallas import tpu_sc as plsc

D = 128
G = 128
LANES = 16
NC = 2
NS = 16
NW = NC * NS
SEGS_PER_W = G // NW
TILE = 224
NBUF = 4
UNROLL = 4
STARTS_PAD = 144
NEG_INF = float("-inf")
NB = D // LANES
SB = STARTS_PAD // LANES


def _scan_boundaries(n_rows, batch_hbm, sid, idbuf, lstarts, shared, merged,
                     starts_v):
    chunk = n_rows // NS
    bufp = idbuf.shape[0]
    iota = lax.iota(jnp.int32, LANES)

    for b in range(SB):
        lstarts[pl.ds(b * LANES, LANES)] = jnp.full((LANES,), n_rows,
                                                    jnp.int32)
    lo_i = jnp.maximum(sid * chunk, 1)
    hi_i = (sid + 1) * chunk
    ab = jnp.minimum(((lo_i - 1) // 8) * 8, n_rows - bufp)
    pltpu.sync_copy(batch_hbm.at[pl.ds(ab, bufp)], idbuf)

    @pl.when(sid == 0)
    def _():
        v0 = idbuf[pl.ds(0, LANES)]
        plsc.store_scatter(lstarts, [v0], jnp.zeros((LANES,), jnp.int32),
                           mask=iota == 0)

    n_iter = -(-chunk // LANES)

    def scan_step(it, carry):
        i0 = lo_i + it * LANES
        li = i0 - ab
        v = idbuf[pl.ds(li, LANES)]
        vp = idbuf[pl.ds(li - 1, LANES)]
        changed = (v != vp) & (iota + i0 < hi_i)
        plsc.store_scatter(lstarts, [v], iota + i0, mask=changed)
        return carry

    lax.fori_loop(0, n_iter, scan_step, 0)

    pltpu.sync_copy(lstarts, shared.at[sid])
    plsc.subcore_barrier()
    pltpu.sync_copy(shared, merged)
    mins = [merged[0, pl.ds(b * LANES, LANES)] for b in range(SB)]
    for r in range(1, NS):
        for b in range(SB):
            mins[b] = jnp.minimum(mins[b], merged[r, pl.ds(b * LANES, LANES)])

    carry = jnp.int32(n_rows)
    for b in reversed(range(SB)):
        r = lax.rev(mins[b], (0,))
        sm = lax.rev(jnp.negative(plsc.cummax(jnp.negative(r))), (0,))
        sm = jnp.minimum(sm, carry)
        starts_v[pl.ds(b * LANES, LANES)] = sm
        carry = sm[0]


def _seg_max_body(n_rows, x_hbm, batch_hbm, out_hbm, idbuf, lstarts, shared,
                  merged, starts_v, buf0, buf1, buf2, buf3, arow,
                  sem0, sem1, sem2, sem3):
    cid = lax.axis_index("c")
    sid = lax.axis_index("s")
    wid = sid * NC + cid

    _scan_boundaries(n_rows, batch_hbm, sid, idbuf, lstarts, shared, merged,
                     starts_v)

    bufs = (buf0, buf1, buf2, buf3)
    sems = (sem0, sem1, sem2, sem3)
    g0 = wid * SEGS_PER_W

    sv = [starts_v[pl.ds(g0 + k, LANES)][0] for k in range(SEGS_PER_W + 1)]
    lo_all = sv[0]
    nt = (sv[SEGS_PER_W] - lo_all + TILE - 1) // TILE

    def tbase_of(t):
        return jnp.minimum(lo_all + t * TILE, n_rows - TILE)

    for k in range(SEGS_PER_W):
        for j in range(NB):
            arow[k, pl.ds(j * LANES, LANES)] = jnp.full((LANES,), NEG_INF,
                                                        jnp.float32)

    for b in range(NBUF - 1):
        @pl.when(b < nt)
        def _():
            pltpu.async_copy(x_hbm.at[pl.ds(tbase_of(b), TILE)], bufs[b],
                             sems[b])

    def tile_step(parity, t):
        buf, sem = bufs[parity], sems[parity]
        pltpu.make_async_copy(
            x_hbm.at[pl.ds(tbase_of(t), TILE)], buf, sem).wait()

        nparity = (parity + NBUF - 1) % NBUF

        @pl.when(t + NBUF - 1 < nt)
        def _():
            pltpu.async_copy(
                x_hbm.at[pl.ds(tbase_of(t + NBUF - 1), TILE)],
                bufs[nparity], sems[nparity])

        tbase = tbase_of(t)
        neg = jnp.full((LANES,), NEG_INF, jnp.float32)

        for k in range(SEGS_PER_W):
            lo = jnp.maximum(sv[k] - tbase, 0)
            hi = jnp.minimum(sv[k + 1] - tbase, TILE)

            @pl.when(hi > lo)
            def _():
                acc = [arow[k, pl.ds(j * LANES, LANES)] for j in range(NB)]

                def rows(rr, acc):
                    out = list(acc)
                    for u in range(UNROLL):
                        i = rr * UNROLL + u
                        m = (i >= lo) & (i < hi)
                        for j in range(NB):
                            v = jnp.where(m, buf[i, pl.ds(j * LANES, LANES)],
                                          neg)
                            out[j] = jnp.maximum(out[j], v)
                    return out

                acc = lax.fori_loop(0, TILE // UNROLL, rows, acc)
                for j in range(NB):
                    arow[k, pl.ds(j * LANES, LANES)] = acc[j]

    def ring_body(p, carry):
        for b in range(NBUF):
            t = NBUF * p + b

            @pl.when(t < nt)
            def _():
                tile_step(b, t)
        return carry

    lax.fori_loop(0, (nt + NBUF - 1) // NBUF, ring_body, 0)
    for k in range(SEGS_PER_W):
        pltpu.sync_copy(arow.at[k], out_hbm.at[g0 + k])


@jax.jit
def kernel(x, batch):
    n_rows = x.shape[0]
    chunk = n_rows // NS
    bufp = ((chunk + LANES + 14) // 8) * 8
    mesh = plsc.VectorSubcoreMesh(core_axis_name="c", subcore_axis_name="s")
    return pl.kernel(
        functools.partial(_seg_max_body, n_rows),
        out_type=jax.ShapeDtypeStruct((G, D), jnp.float32),
        mesh=mesh,
        compiler_params=pltpu.CompilerParams(
            use_tc_tiling_on_sc=False, needs_layout_passes=False),
        scratch_types=[
            pltpu.VMEM((bufp,), jnp.int32),
            pltpu.VMEM((STARTS_PAD,), jnp.int32),
            pltpu.VMEM_SHARED((NS, STARTS_PAD), jnp.int32),
            pltpu.VMEM((NS, STARTS_PAD), jnp.int32),
            pltpu.VMEM((STARTS_PAD,), jnp.int32),
            pltpu.VMEM((TILE, D), jnp.float32),
            pltpu.VMEM((TILE, D), jnp.float32),
            pltpu.VMEM((TILE, D), jnp.float32),
            pltpu.VMEM((TILE, D), jnp.float32),
            pltpu.VMEM((SEGS_PER_W, D), jnp.float32),
            pltpu.SemaphoreType.DMA,
            pltpu.SemaphoreType.DMA,
            pltpu.SemaphoreType.DMA,
            pltpu.SemaphoreType.DMA,
        ],
    )(x, batch)


# speedup vs baseline: 1.0096x; 1.0096x over previous
import functools

import jax
import jax.numpy as jnp
from jax import lax
from jax.experimental import pallas as pl
from jax.experimental.pallas import tpu as pltpu
from jax.experimental.pallas import tpu_sc as plsc

D = 128
G = 128
LANES = 16
NC = 2
NS = 16
NW = NC * NS
SEGS_PER_W = G // NW
TILE = 288
NBUF = 3
UNROLL = 4
STARTS_PAD = 144
NEG_INF = float("-inf")
NB = D // LANES
SB = STARTS_PAD // LANES


def _scan_boundaries(n_rows, batch_hbm, sid, idbuf, lstarts, shared, merged,
                     starts_v):
    chunk = n_rows // NS
    bufp = idbuf.shape[0]
    iota = lax.iota(jnp.int32, LANES)

    for b in range(SB):
        lstarts[pl.ds(b * LANES, LANES)] = jnp.full((LANES,), n_rows,
                                                    jnp.int32)
    lo_i = jnp.maximum(sid * chunk, 1)
    hi_i = (sid + 1) * chunk
    ab = jnp.minimum(((lo_i - 1) // 8) * 8, n_rows - bufp)
    pltpu.sync_copy(batch_hbm.at[pl.ds(ab, bufp)], idbuf)

    @pl.when(sid == 0)
    def _():
        v0 = idbuf[pl.ds(0, LANES)]
        plsc.store_scatter(lstarts, [v0], jnp.zeros((LANES,), jnp.int32),
                           mask=iota == 0)

    n_iter = -(-chunk // LANES)

    def scan_step(it, carry):
        i0 = lo_i + it * LANES
        li = i0 - ab
        v = idbuf[pl.ds(li, LANES)]
        vp = idbuf[pl.ds(li - 1, LANES)]
        changed = (v != vp) & (iota + i0 < hi_i)
        plsc.store_scatter(lstarts, [v], iota + i0, mask=changed)
        return carry

    lax.fori_loop(0, n_iter, scan_step, 0)

    pltpu.sync_copy(lstarts, shared.at[sid])
    plsc.subcore_barrier()
    pltpu.sync_copy(shared, merged)
    mins = [merged[0, pl.ds(b * LANES, LANES)] for b in range(SB)]
    for r in range(1, NS):
        for b in range(SB):
            mins[b] = jnp.minimum(mins[b], merged[r, pl.ds(b * LANES, LANES)])

    carry = jnp.int32(n_rows)
    for b in reversed(range(SB)):
        r = lax.rev(mins[b], (0,))
        sm = lax.rev(jnp.negative(plsc.cummax(jnp.negative(r))), (0,))
        sm = jnp.minimum(sm, carry)
        starts_v[pl.ds(b * LANES, LANES)] = sm
        carry = sm[0]


def _seg_max_body(n_rows, x_hbm, batch_hbm, out_hbm, idbuf, lstarts, shared,
                  merged, starts_v, buf0, buf1, buf2, arow, sem0, sem1, sem2):
    cid = lax.axis_index("c")
    sid = lax.axis_index("s")
    wid = sid * NC + cid

    _scan_boundaries(n_rows, batch_hbm, sid, idbuf, lstarts, shared, merged,
                     starts_v)

    bufs = (buf0, buf1, buf2)
    sems = (sem0, sem1, sem2)
    g0 = wid * SEGS_PER_W

    sv = [starts_v[pl.ds(g0 + k, LANES)][0] for k in range(SEGS_PER_W + 1)]
    lo_all = sv[0]
    nt = (sv[SEGS_PER_W] - lo_all + TILE - 1) // TILE

    def tbase_of(t):
        return jnp.minimum(lo_all + t * TILE, n_rows - TILE)

    for k in range(SEGS_PER_W):
        for j in range(NB):
            arow[k, pl.ds(j * LANES, LANES)] = jnp.full((LANES,), NEG_INF,
                                                        jnp.float32)

    for b in range(NBUF - 1):
        @pl.when(b < nt)
        def _():
            pltpu.async_copy(x_hbm.at[pl.ds(tbase_of(b), TILE)], bufs[b],
                             sems[b])

    def tile_step(parity, t):
        buf, sem = bufs[parity], sems[parity]
        pltpu.make_async_copy(
            x_hbm.at[pl.ds(tbase_of(t), TILE)], buf, sem).wait()

        nparity = (parity + NBUF - 1) % NBUF

        @pl.when(t + NBUF - 1 < nt)
        def _():
            pltpu.async_copy(
                x_hbm.at[pl.ds(tbase_of(t + NBUF - 1), TILE)],
                bufs[nparity], sems[nparity])

        tbase = tbase_of(t)
        neg = jnp.full((LANES,), NEG_INF, jnp.float32)

        for k in range(SEGS_PER_W):
            lo = jnp.maximum(sv[k] - tbase, 0)
            hi = jnp.minimum(sv[k + 1] - tbase, TILE)

            @pl.when(hi > lo)
            def _():
                acc = [arow[k, pl.ds(j * LANES, LANES)] for j in range(NB)]

                def rows(rr, acc):
                    out = list(acc)
                    for u in range(UNROLL):
                        i = rr * UNROLL + u
                        m = (i >= lo) & (i < hi)
                        for j in range(NB):
                            v = jnp.where(m, buf[i, pl.ds(j * LANES, LANES)],
                                          neg)
                            out[j] = jnp.maximum(out[j], v)
                    return out

                acc = lax.fori_loop(0, TILE // UNROLL, rows, acc)
                for j in range(NB):
                    arow[k, pl.ds(j * LANES, LANES)] = acc[j]

    def ring_body(p, carry):
        for b in range(NBUF):
            t = NBUF * p + b

            @pl.when(t < nt)
            def _():
                tile_step(b, t)
        return carry

    lax.fori_loop(0, (nt + NBUF - 1) // NBUF, ring_body, 0)
    for k in range(SEGS_PER_W):
        pltpu.sync_copy(arow.at[k], out_hbm.at[g0 + k])


@jax.jit
def kernel(x, batch):
    n_rows = x.shape[0]
    chunk = n_rows // NS
    bufp = ((chunk + LANES + 14) // 8) * 8
    mesh = plsc.VectorSubcoreMesh(core_axis_name="c", subcore_axis_name="s")
    return pl.kernel(
        functools.partial(_seg_max_body, n_rows),
        out_type=jax.ShapeDtypeStruct((G, D), jnp.float32),
        mesh=mesh,
        compiler_params=pltpu.CompilerParams(
            use_tc_tiling_on_sc=False, needs_layout_passes=False),
        scratch_types=[
            pltpu.VMEM((bufp,), jnp.int32),
            pltpu.VMEM((STARTS_PAD,), jnp.int32),
            pltpu.VMEM_SHARED((NS, STARTS_PAD), jnp.int32),
            pltpu.VMEM((NS, STARTS_PAD), jnp.int32),
            pltpu.VMEM((STARTS_PAD,), jnp.int32),
            pltpu.VMEM((TILE, D), jnp.float32),
            pltpu.VMEM((TILE, D), jnp.float32),
            pltpu.VMEM((TILE, D), jnp.float32),
            pltpu.VMEM((SEGS_PER_W, D), jnp.float32),
            pltpu.SemaphoreType.DMA,
            pltpu.SemaphoreType.DMA,
            pltpu.SemaphoreType.DMA,
        ],
    )(x, batch)


# speedup vs baseline: 1.0244x; 1.0146x over previous
import functools

import jax
import jax.numpy as jnp
from jax import lax
from jax.experimental import pallas as pl
from jax.experimental.pallas import tpu as pltpu
from jax.experimental.pallas import tpu_sc as plsc

D = 128
G = 128
LANES = 16
NC = 2
NS = 16
NW = NC * NS
SEGS_PER_W = G // NW
TILE = 256
NBUF = 3
UNROLL = 4
STARTS_PAD = 144
NEG_INF = float("-inf")
NB = D // LANES
SB = STARTS_PAD // LANES


def _scan_boundaries(n_rows, batch_hbm, sid, idbuf, lstarts, shared, merged,
                     starts_v):
    chunk = n_rows // NS
    bufp = idbuf.shape[0]
    iota = lax.iota(jnp.int32, LANES)

    for b in range(SB):
        lstarts[pl.ds(b * LANES, LANES)] = jnp.full((LANES,), n_rows,
                                                    jnp.int32)
    lo_i = jnp.maximum(sid * chunk, 1)
    hi_i = (sid + 1) * chunk
    ab = jnp.minimum(((lo_i - 1) // 8) * 8, n_rows - bufp)
    pltpu.sync_copy(batch_hbm.at[pl.ds(ab, bufp)], idbuf)

    @pl.when(sid == 0)
    def _():
        v0 = idbuf[pl.ds(0, LANES)]
        plsc.store_scatter(lstarts, [v0], jnp.zeros((LANES,), jnp.int32),
                           mask=iota == 0)

    n_iter = -(-chunk // (2 * LANES))

    def scan_step(it, carry):
        for h in range(2):
            i0 = lo_i + (2 * it + h) * LANES
            li = i0 - ab
            v = idbuf[pl.ds(li, LANES)]
            vp = idbuf[pl.ds(li - 1, LANES)]
            changed = (v != vp) & (iota + i0 < hi_i)
            plsc.store_scatter(lstarts, [v], iota + i0, mask=changed)
        return carry

    lax.fori_loop(0, n_iter, scan_step, 0)

    pltpu.sync_copy(lstarts, shared.at[sid])
    plsc.subcore_barrier()
    pltpu.sync_copy(shared, merged)
    mins = [merged[0, pl.ds(b * LANES, LANES)] for b in range(SB)]
    for r in range(1, NS):
        for b in range(SB):
            mins[b] = jnp.minimum(mins[b], merged[r, pl.ds(b * LANES, LANES)])

    carry = jnp.int32(n_rows)
    for b in reversed(range(SB)):
        r = lax.rev(mins[b], (0,))
        sm = lax.rev(jnp.negative(plsc.cummax(jnp.negative(r))), (0,))
        sm = jnp.minimum(sm, carry)
        starts_v[pl.ds(b * LANES, LANES)] = sm
        carry = sm[0]


def _seg_max_body(n_rows, x_hbm, batch_hbm, out_hbm, idbuf, lstarts, shared,
                  merged, starts_v, buf0, buf1, buf2, arow, sem0, sem1, sem2):
    cid = lax.axis_index("c")
    sid = lax.axis_index("s")
    wid = sid * NC + cid

    _scan_boundaries(n_rows, batch_hbm, sid, idbuf, lstarts, shared, merged,
                     starts_v)

    bufs = (buf0, buf1, buf2)
    sems = (sem0, sem1, sem2)
    g0 = wid * SEGS_PER_W

    sv = [starts_v[pl.ds(g0 + k, LANES)][0] for k in range(SEGS_PER_W + 1)]
    lo_all = sv[0]
    nt = (sv[SEGS_PER_W] - lo_all + TILE - 1) // TILE

    def tbase_of(t):
        return jnp.minimum(lo_all + t * TILE, n_rows - TILE)

    for k in range(SEGS_PER_W):
        for j in range(NB):
            arow[k, pl.ds(j * LANES, LANES)] = jnp.full((LANES,), NEG_INF,
                                                        jnp.float32)

    for b in range(NBUF - 1):
        @pl.when(b < nt)
        def _():
            pltpu.async_copy(x_hbm.at[pl.ds(tbase_of(b), TILE)], bufs[b],
                             sems[b])

    def tile_step(parity, t):
        buf, sem = bufs[parity], sems[parity]
        pltpu.make_async_copy(
            x_hbm.at[pl.ds(tbase_of(t), TILE)], buf, sem).wait()

        nparity = (parity + NBUF - 1) % NBUF

        @pl.when(t + NBUF - 1 < nt)
        def _():
            pltpu.async_copy(
                x_hbm.at[pl.ds(tbase_of(t + NBUF - 1), TILE)],
                bufs[nparity], sems[nparity])

        tbase = tbase_of(t)
        neg = jnp.full((LANES,), NEG_INF, jnp.float32)

        for k in range(SEGS_PER_W):
            lo = jnp.maximum(sv[k] - tbase, 0)
            hi = jnp.minimum(sv[k + 1] - tbase, TILE)

            @pl.when(hi > lo)
            def _():
                acc = [arow[k, pl.ds(j * LANES, LANES)] for j in range(NB)]

                def rows(rr, acc):
                    out = list(acc)
                    for u in range(UNROLL):
                        i = rr * UNROLL + u
                        m = (i >= lo) & (i < hi)
                        for j in range(NB):
                            v = jnp.where(m, buf[i, pl.ds(j * LANES, LANES)],
                                          neg)
                            out[j] = jnp.maximum(out[j], v)
                    return out

                acc = lax.fori_loop(0, TILE // UNROLL, rows, acc)
                for j in range(NB):
                    arow[k, pl.ds(j * LANES, LANES)] = acc[j]

    def ring_body(p, carry):
        for b in range(NBUF):
            t = NBUF * p + b

            @pl.when(t < nt)
            def _():
                tile_step(b, t)
        return carry

    lax.fori_loop(0, (nt + NBUF - 1) // NBUF, ring_body, 0)
    for k in range(SEGS_PER_W):
        pltpu.sync_copy(arow.at[k], out_hbm.at[g0 + k])


@jax.jit
def kernel(x, batch):
    n_rows = x.shape[0]
    chunk = n_rows // NS
    bufp = ((chunk + 2 * LANES + 14) // 8) * 8
    mesh = plsc.VectorSubcoreMesh(core_axis_name="c", subcore_axis_name="s")
    return pl.kernel(
        functools.partial(_seg_max_body, n_rows),
        out_type=jax.ShapeDtypeStruct((G, D), jnp.float32),
        mesh=mesh,
        compiler_params=pltpu.CompilerParams(
            use_tc_tiling_on_sc=False, needs_layout_passes=False),
        scratch_types=[
            pltpu.VMEM((bufp,), jnp.int32),
            pltpu.VMEM((STARTS_PAD,), jnp.int32),
            pltpu.VMEM_SHARED((NS, STARTS_PAD), jnp.int32),
            pltpu.VMEM((NS, STARTS_PAD), jnp.int32),
            pltpu.VMEM((STARTS_PAD,), jnp.int32),
            pltpu.VMEM((TILE, D), jnp.float32),
            pltpu.VMEM((TILE, D), jnp.float32),
            pltpu.VMEM((TILE, D), jnp.float32),
            pltpu.VMEM((SEGS_PER_W, D), jnp.float32),
            pltpu.SemaphoreType.DMA,
            pltpu.SemaphoreType.DMA,
            pltpu.SemaphoreType.DMA,
        ],
    )(x, batch)


# speedup vs baseline: 1.0389x; 1.0142x over previous
import functools

import jax
import jax.numpy as jnp
from jax import lax
from jax.experimental import pallas as pl
from jax.experimental.pallas import tpu as pltpu
from jax.experimental.pallas import tpu_sc as plsc

D = 128
G = 128
LANES = 16
NC = 2
NS = 16
NW = NC * NS
SEGS_PER_W = G // NW
TILE = 256
NBUF = 3
UNROLL = 4
STARTS_PAD = 144
NEG_INF = float("-inf")
NB = D // LANES
SB = STARTS_PAD // LANES


def _scan_boundaries(n_rows, batch_hbm, sid, idbuf, lstarts, shared, merged,
                     starts_v):
    chunk = n_rows // NS
    bufp = idbuf.shape[0]
    iota = lax.iota(jnp.int32, LANES)

    for b in range(SB):
        lstarts[pl.ds(b * LANES, LANES)] = jnp.full((LANES,), n_rows,
                                                    jnp.int32)
    lo_i = jnp.maximum(sid * chunk, 1)
    hi_i = (sid + 1) * chunk
    ab = jnp.minimum(((lo_i - 1) // 8) * 8, n_rows - bufp)
    pltpu.sync_copy(batch_hbm.at[pl.ds(ab, bufp)], idbuf)

    @pl.when(sid == 0)
    def _():
        v0 = idbuf[pl.ds(0, LANES)]
        plsc.store_scatter(lstarts, [v0], jnp.zeros((LANES,), jnp.int32),
                           mask=iota == 0)

    n_iter = -(-chunk // (2 * LANES))

    def scan_step(it, carry):
        for h in range(2):
            i0 = lo_i + (2 * it + h) * LANES
            li = i0 - ab
            v = idbuf[pl.ds(li, LANES)]
            vp = idbuf[pl.ds(li - 1, LANES)]
            changed = (v != vp) & (iota + i0 < hi_i)
            plsc.store_scatter(lstarts, [v], iota + i0, mask=changed)
        return carry

    lax.fori_loop(0, n_iter, scan_step, 0)

    pltpu.sync_copy(lstarts, shared.at[sid])
    plsc.subcore_barrier()
    pltpu.sync_copy(shared, merged)
    mins = [merged[0, pl.ds(b * LANES, LANES)] for b in range(SB)]
    for r in range(1, NS):
        for b in range(SB):
            mins[b] = jnp.minimum(mins[b], merged[r, pl.ds(b * LANES, LANES)])

    carry = jnp.int32(n_rows)
    for b in reversed(range(SB)):
        r = lax.rev(mins[b], (0,))
        sm = lax.rev(jnp.negative(plsc.cummax(jnp.negative(r))), (0,))
        sm = jnp.minimum(sm, carry)
        starts_v[pl.ds(b * LANES, LANES)] = sm
        carry = sm[0]


def _seg_max_body(n_rows, x_hbm, batch_hbm, out_hbm, idbuf, lstarts, shared,
                  merged, starts_v, buf0, buf1, buf2, arow, sem0, sem1, sem2):
    cid = lax.axis_index("c")
    sid = lax.axis_index("s")
    wid = sid * NC + cid

    _scan_boundaries(n_rows, batch_hbm, sid, idbuf, lstarts, shared, merged,
                     starts_v)

    bufs = (buf0, buf1, buf2)
    sems = (sem0, sem1, sem2)
    g0 = wid * SEGS_PER_W

    sv = [starts_v[pl.ds(g0 + k, LANES)][0] for k in range(SEGS_PER_W + 1)]
    lo_all = sv[0]
    nt = (sv[SEGS_PER_W] - lo_all + TILE - 1) // TILE

    def tbase_of(t):
        return jnp.minimum(lo_all + t * TILE, n_rows - TILE)

    for k in range(SEGS_PER_W):
        for j in range(NB):
            arow[k, pl.ds(j * LANES, LANES)] = jnp.full((LANES,), NEG_INF,
                                                        jnp.float32)

    for b in range(NBUF - 1):
        @pl.when(b < nt)
        def _():
            pltpu.async_copy(x_hbm.at[pl.ds(tbase_of(b), TILE)], bufs[b],
                             sems[b])

    def tile_step(parity, t):
        buf, sem = bufs[parity], sems[parity]
        pltpu.make_async_copy(
            x_hbm.at[pl.ds(tbase_of(t), TILE)], buf, sem).wait()

        nparity = (parity + NBUF - 1) % NBUF

        @pl.when(t + NBUF - 1 < nt)
        def _():
            pltpu.async_copy(
                x_hbm.at[pl.ds(tbase_of(t + NBUF - 1), TILE)],
                bufs[nparity], sems[nparity])

        tbase = tbase_of(t)
        neg = jnp.full((LANES,), NEG_INF, jnp.float32)

        def seg_body(k, carry):
            lo = jnp.maximum(
                starts_v[pl.ds(g0 + k, LANES)][0] - tbase, 0)
            hi = jnp.minimum(
                starts_v[pl.ds(g0 + k + 1, LANES)][0] - tbase, TILE)

            @pl.when(hi > lo)
            def _():
                acc = [arow[k, pl.ds(j * LANES, LANES)] for j in range(NB)]

                def rows(rr, acc):
                    out = list(acc)
                    for u in range(UNROLL):
                        i = rr * UNROLL + u
                        m = (i >= lo) & (i < hi)
                        for j in range(NB):
                            v = jnp.where(m, buf[i, pl.ds(j * LANES, LANES)],
                                          neg)
                            out[j] = jnp.maximum(out[j], v)
                    return out

                acc = lax.fori_loop(0, TILE // UNROLL, rows, acc)
                for j in range(NB):
                    arow[k, pl.ds(j * LANES, LANES)] = acc[j]

            return carry

        lax.fori_loop(0, SEGS_PER_W, seg_body, 0)

    def ring_body(p, carry):
        for b in range(NBUF):
            t = NBUF * p + b

            @pl.when(t < nt)
            def _():
                tile_step(b, t)
        return carry

    lax.fori_loop(0, (nt + NBUF - 1) // NBUF, ring_body, 0)
    for k in range(SEGS_PER_W):
        pltpu.sync_copy(arow.at[k], out_hbm.at[g0 + k])


@jax.jit
def kernel(x, batch):
    n_rows = x.shape[0]
    chunk = n_rows // NS
    bufp = ((chunk + 2 * LANES + 14) // 8) * 8
    mesh = plsc.VectorSubcoreMesh(core_axis_name="c", subcore_axis_name="s")
    return pl.kernel(
        functools.partial(_seg_max_body, n_rows),
        out_type=jax.ShapeDtypeStruct((G, D), jnp.float32),
        mesh=mesh,
        compiler_params=pltpu.CompilerParams(
            use_tc_tiling_on_sc=False, needs_layout_passes=False),
        scratch_types=[
            pltpu.VMEM((bufp,), jnp.int32),
            pltpu.VMEM((STARTS_PAD,), jnp.int32),
            pltpu.VMEM_SHARED((NS, STARTS_PAD), jnp.int32),
            pltpu.VMEM((NS, STARTS_PAD), jnp.int32),
            pltpu.VMEM((STARTS_PAD,), jnp.int32),
            pltpu.VMEM((TILE, D), jnp.float32),
            pltpu.VMEM((TILE, D), jnp.float32),
            pltpu.VMEM((TILE, D), jnp.float32),
            pltpu.VMEM((SEGS_PER_W, D), jnp.float32),
            pltpu.SemaphoreType.DMA,
            pltpu.SemaphoreType.DMA,
            pltpu.SemaphoreType.DMA,
        ],
    )(x, batch)
